# Initial kernel scaffold; baseline (speedup 1.0000x reference)
import jax
import jax.numpy as jnp
from jax.experimental import pallas as pl


def kernel(x, e, edge_index, inverse_edge_index, Wn, bn, Wt, bt, We, be, WU, bU, WVf, bVf, WVt, bVt, WiU, biU, W_ph, g_node, b_node, g_edge, b_edge):
    raise NotImplementedError("write your pallas kernel here")


# speedup vs baseline: 5.4897x; 5.4897x over previous
import functools

import jax
import jax.numpy as jnp
from jax import lax
from jax.experimental import pallas as pl
from jax.experimental.pallas import tpu as pltpu
from jax.experimental.pallas import tpu_sc as plsc

_B = 2
_N = 10000
_SF = 20
_H = 128
_L = _N * _SF
_EB = 1600
_NBN = _EB // _SF
_NBLK = _L // _EB
_CB = 512
_NCH = (_B * _L + _CB - 1) // _CB
_PADR = _NCH * _CB
_LI = _L + _EB

_pallas_call = pl.pallas_call


def _dotT(a, w):
    return lax.dot_general(a, w, (((1,), (1,)), ((), ())),
                           preferred_element_type=jnp.float32)


def _k1_body(x_ref, wn_ref, bn_ref, wt_ref, bt_ref, ux_ref, vx_ref):
    x = x_ref[0]
    ux_ref[0] = _dotT(x, wn_ref[...]) + bn_ref[...]
    vx_ref[0] = _dotT(x, wt_ref[...]) + bt_ref[...]


def _run_k1(x, Wn, bn2, Wt, bt2):
    full = pl.BlockSpec((_H, _H), lambda b: (0, 0))
    row = pl.BlockSpec((1, _H), lambda b: (0, 0))
    return _pallas_call(
        _k1_body,
        grid=(_B,),
        in_specs=[pl.BlockSpec((1, _N, _H), lambda b: (b, 0, 0)),
                  full, row, full, row],
        out_specs=[pl.BlockSpec((1, _N, _H), lambda b: (b, 0, 0))] * 2,
        out_shape=[jax.ShapeDtypeStruct((_B, _N, _H), jnp.float32)] * 2,
    )(x, Wn, bn2, Wt, bt2)


def _k2_body(e_ref, we_ref, be_ref, wu_ref, bu_ref, wi_ref, bi_ref, wph_ref,
             ve_ref, ue_ref, iue_ref):
    i = pl.program_id(1)
    eb = e_ref[0]
    z = _dotT(eb, we_ref[...]) + be_ref[...]
    z3 = z.reshape(_NBN, _SF, _H)
    m = jnp.max(z3, axis=1, keepdims=True)
    ex = jnp.exp(z3 - m)
    s = jnp.sum(ex, axis=1, keepdims=True)
    ve_ref[0] = (ex / s).reshape(_EB, _H)
    ue_ref[0] = _dotT(eb, wu_ref[...]) + bu_ref[...]
    iue = _dotT(eb, wi_ref[...]) + bi_ref[...]
    wph_rows = jnp.broadcast_to(wph_ref[...], (_EB, _H))
    iue_ref[0] = jnp.where(i == _NBLK, wph_rows, iue)


def _run_k2(e, We, be2, WU, bU2, WiU, biU2, Wph2):
    full = pl.BlockSpec((_H, _H), lambda b, i: (0, 0))
    row = pl.BlockSpec((1, _H), lambda b, i: (0, 0))
    eblk = pl.BlockSpec((1, _EB, _H),
                        lambda b, i: (b, jnp.minimum(i, _NBLK - 1), 0))
    return _pallas_call(
        _k2_body,
        grid=(_B, _NBLK + 1),
        in_specs=[eblk, full, row, full, row, full, row, row],
        out_specs=[eblk, eblk,
                   pl.BlockSpec((1, _EB, _H), lambda b, i: (b, i, 0))],
        out_shape=[jax.ShapeDtypeStruct((_B, _L, _H), jnp.float32),
                   jax.ShapeDtypeStruct((_B, _L, _H), jnp.float32),
                   jax.ShapeDtypeStruct((_B, _LI, _H), jnp.float32)],
    )(e, We, be2, WU, bU2, WiU, biU2, Wph2)


def _k4_body(ve_ref, vxg_ref, ux_ref, xt_ref):
    prod = ve_ref[0] * vxg_ref[...]
    to = jnp.sum(prod.reshape(_NBN, _SF, _H), axis=1)
    xt_ref[0] = ux_ref[0] + to


def _run_k4(ve, vxg_flat, ux):
    return _pallas_call(
        _k4_body,
        grid=(_B, _NBLK),
        in_specs=[pl.BlockSpec((1, _EB, _H), lambda b, i: (b, i, 0)),
                  pl.BlockSpec((_EB, _H), lambda b, i: (b * _NBLK + i, 0)),
                  pl.BlockSpec((1, _NBN, _H), lambda b, i: (b, i, 0))],
        out_specs=pl.BlockSpec((1, _NBN, _H), lambda b, i: (b, i, 0)),
        out_shape=jax.ShapeDtypeStruct((_B, _N, _H), jnp.float32),
    )(ve, vxg_flat, ux)


def _stats_body(v_ref, st_ref):
    first = jnp.logical_and(pl.program_id(0) == 0, pl.program_id(1) == 0)

    @pl.when(first)
    def _():
        st_ref[...] = jnp.zeros((8, _H), jnp.float32)

    v = v_ref[0].reshape(-1, _H)
    s = jnp.sum(v, axis=0, keepdims=True)
    ss = jnp.sum(v * v, axis=0, keepdims=True)
    st_ref[...] += jnp.concatenate([s, ss, jnp.zeros((6, _H), jnp.float32)], 0)


def _run_stats(arr, blk_rows, nblk):
    return _pallas_call(
        _stats_body,
        grid=(_B, nblk),
        in_specs=[pl.BlockSpec((1, blk_rows, _H), lambda b, i: (b, i, 0))],
        out_specs=pl.BlockSpec((8, _H), lambda b, i: (0, 0)),
        out_shape=jax.ShapeDtypeStruct((8, _H), jnp.float32),
    )(arr)


def _k5b_body(x_ref, xt_ref, st_ref, g_ref, bb_ref, wvf_ref, bvf_ref,
              wvt_ref, bvt_ref, xn_ref, vxf_ref, vxt_ref):
    cnt = jnp.float32(_B * _N)
    m = st_ref[0:1, :] / cnt
    v = st_ref[1:2, :] / cnt - m * m
    xt = xt_ref[0]
    norm = (xt - m) * lax.rsqrt(v + 1e-5) * g_ref[...] + bb_ref[...]
    xn = x_ref[0] + jnp.maximum(norm, 0.0)
    xn_ref[0] = xn
    vxf_ref[0] = _dotT(xn, wvf_ref[...]) + bvf_ref[...]
    vxt_ref[0] = _dotT(xn, wvt_ref[...]) + bvt_ref[...]


def _run_k5b(x, x_tmp, st, g2, b2, WVf, bVf2, WVt, bVt2):
    full = pl.BlockSpec((_H, _H), lambda b, i: (0, 0))
    row = pl.BlockSpec((1, _H), lambda b, i: (0, 0))
    nblk = pl.BlockSpec((1, _NBN, _H), lambda b, i: (b, i, 0))
    stat = pl.BlockSpec((8, _H), lambda b, i: (0, 0))
    return _pallas_call(
        _k5b_body,
        grid=(_B, _N // _NBN),
        in_specs=[nblk, nblk, stat, row, row, full, row, full, row],
        out_specs=[nblk] * 3,
        out_shape=[jax.ShapeDtypeStruct((_B, _N, _H), jnp.float32)] * 3,
    )(x, x_tmp, st, g2, b2, WVf, bVf2, WVt, bVt2)


def _k7_body(ue_ref, vxeg_ref, inv_ref, vxf_ref, et_ref, st_ref):
    first = jnp.logical_and(pl.program_id(0) == 0, pl.program_id(1) == 0)

    @pl.when(first)
    def _():
        st_ref[...] = jnp.zeros((8, _H), jnp.float32)

    vxe = vxeg_ref[...].reshape(_NBN, _SF, _H) + vxf_ref[0].reshape(_NBN, 1, _H)
    et = ue_ref[0] + vxe.reshape(_EB, _H) + inv_ref[...]
    et_ref[0] = et
    s = jnp.sum(et, axis=0, keepdims=True)
    ss = jnp.sum(et * et, axis=0, keepdims=True)
    st_ref[...] += jnp.concatenate([s, ss, jnp.zeros((6, _H), jnp.float32)], 0)


def _run_k7(ue, vxeg_flat, inv_flat, vxf):
    return _pallas_call(
        _k7_body,
        grid=(_B, _NBLK),
        in_specs=[pl.BlockSpec((1, _EB, _H), lambda b, i: (b, i, 0)),
                  pl.BlockSpec((_EB, _H), lambda b, i: (b * _NBLK + i, 0)),
                  pl.BlockSpec((_EB, _H), lambda b, i: (b * _NBLK + i, 0)),
                  pl.BlockSpec((1, _NBN, _H), lambda b, i: (b, i, 0))],
        out_specs=[pl.BlockSpec((1, _EB, _H), lambda b, i: (b, i, 0)),
                   pl.BlockSpec((8, _H), lambda b, i: (0, 0))],
        out_shape=[jax.ShapeDtypeStruct((_B, _L, _H), jnp.float32),
                   jax.ShapeDtypeStruct((8, _H), jnp.float32)],
    )(ue, vxeg_flat, inv_flat, vxf)


def _k8_body(e_ref, et_ref, st_ref, g_ref, bb_ref, en_ref):
    cnt = jnp.float32(_B * _L)
    m = st_ref[0:1, :] / cnt
    v = st_ref[1:2, :] / cnt - m * m
    norm = (et_ref[0] - m) * lax.rsqrt(v + 1e-5) * g_ref[...] + bb_ref[...]
    en_ref[0] = e_ref[0] + jnp.maximum(norm, 0.0)


def _run_k8(e, e_tmp, st, g2, b2):
    eblk = pl.BlockSpec((1, _EB, _H), lambda b, i: (b, i, 0))
    return _pallas_call(
        _k8_body,
        grid=(_B, _NBLK),
        in_specs=[eblk, eblk,
                  pl.BlockSpec((8, _H), lambda b, i: (0, 0)),
                  pl.BlockSpec((1, _H), lambda b, i: (0, 0)),
                  pl.BlockSpec((1, _H), lambda b, i: (0, 0))],
        out_specs=eblk,
        out_shape=jax.ShapeDtypeStruct((_B, _L, _H), jnp.float32),
    )(e, e_tmp, st, g2, b2)


def _gather_rows(table, idx_pad):
    info = plsc.get_sparse_core_info()
    nw = info.num_cores * info.num_subcores
    kmax = (_NCH + nw - 1) // nw
    mesh = plsc.VectorSubcoreMesh(core_axis_name="c", subcore_axis_name="s")

    @functools.partial(
        pl.kernel,
        mesh=mesh,
        out_type=jax.ShapeDtypeStruct((_PADR, _H), jnp.float32),
        scratch_types=[pltpu.VMEM((_CB,), jnp.int32),
                       pltpu.VMEM((_CB, _H), jnp.float32),
                       pltpu.SemaphoreType.DMA],
    )
    def k(table_hbm, idx_hbm, out_hbm, idx_v, rows_v, sem):
        wid = lax.axis_index("s") * info.num_cores + lax.axis_index("c")

        def body(kk, carry):
            c = wid + nw * kk

            @pl.when(c < _NCH)
            def _():
                base = c * _CB
                pltpu.sync_copy(idx_hbm.at[pl.ds(base, _CB)], idx_v)
                pltpu.async_copy(table_hbm.at[idx_v], rows_v, sem).wait()
                pltpu.sync_copy(rows_v, out_hbm.at[pl.ds(base, _CB)])

            return carry

        lax.fori_loop(0, kmax, body, 0)

    return k(table, idx_pad)


def kernel(x, e, edge_index, inverse_edge_index, Wn, bn, Wt, bt, We, be,
           WU, bU, WVf, bVf, WVt, bVt, WiU, biU, W_ph,
           g_node, b_node, g_edge, b_edge):
    r = lambda v: v.reshape(1, _H)
    boff = jnp.arange(_B, dtype=jnp.int32)[:, None]
    idx_e = (edge_index.astype(jnp.int32) + boff * _N).reshape(-1)
    idx_e = jnp.pad(idx_e, (0, _PADR - _B * _L))
    idx_inv = (inverse_edge_index.astype(jnp.int32) + boff * _LI).reshape(-1)
    idx_inv = jnp.pad(idx_inv, (0, _PADR - _B * _L))

    ux, vx = _run_k1(x, Wn, r(bn), Wt, r(bt))
    ve, ue, iue = _run_k2(e, We, r(be), WU, r(bU), WiU, r(biU), r(W_ph))

    vxg_flat = _gather_rows(vx.reshape(_B * _N, _H), idx_e)
    inv_flat = _gather_rows(iue.reshape(_B * _LI, _H), idx_inv)

    x_tmp = _run_k4(ve, vxg_flat, ux)
    st_n = _run_stats(x_tmp, _NBN, _N // _NBN)
    x_new, vxf, vxt = _run_k5b(x, x_tmp, st_n, r(g_node), r(b_node),
                               WVf, r(bVf), WVt, r(bVt))

    vxeg_flat = _gather_rows(vxt.reshape(_B * _N, _H), idx_e)

    e_tmp, st_e = _run_k7(ue, vxeg_flat, inv_flat, vxf)
    e_new = _run_k8(e, e_tmp, st_e, r(g_edge), r(b_edge))
    return (x_new, e_new)
